# baseline (speedup 1.0000x reference)
import functools

import jax
import jax.numpy as jnp
from jax import lax
from jax.experimental import pallas as pl
from jax.experimental.pallas import tpu as pltpu
from jax.experimental.pallas import tpu_sc as plsc

N_ITEMS = 8000
N_ATTRS = 2000
N_NODES = N_ITEMS + N_ATTRS
N_EDGES = 160000
IN_DIM = 512
EMBED_DIM = 256
HALF = EMBED_DIM // 2

NC = 2
NS = 16
CHUNK = 128

EROWS = 1280
E_PAD = EROWS * CHUNK - N_EDGES
TROWS = EROWS // NS
NCHUNK = TROWS
ACC_ROWS = N_NODES
ZROWS = 128
GROWS = N_NODES + ZROWS

DEG_EDGES = EROWS * CHUNK
EPT_DEG = DEG_EDGES // (NC * NS)
DEG_N = EPT_DEG // CHUNK
DEG_ACC = N_NODES + 8

ROWS_PER_TILE = 624
ROWS_REM = N_NODES - NS * ROWS_PER_TILE
PAIRS = 4096
GPT = 2 * PAIRS // (NC * NS)

_sc_mesh = plsc.VectorSubcoreMesh(core_axis_name="c", subcore_axis_name="s")


def _dinv_of(deg_ref):
    return lax.rsqrt(deg_ref[:, 0] + deg_ref[:, 1] + 1.0)


def _encg_body(x_ref, w1_ref, b1_ref, w2_ref, b2_ref, wc_ref, deg_ref,
               o_ref, z_scr):
    j = pl.program_id(1)

    @pl.when(j == 0)
    def _():
        h = jnp.dot(x_ref[...].astype(jnp.bfloat16),
                    w1_ref[...].astype(jnp.bfloat16),
                    preferred_element_type=jnp.float32)
        h = h + b1_ref[...]
        h = jnp.where(h > 0, h, jnp.exp(h) - 1.0)
        z_scr[...] = (
            jnp.dot(h.astype(jnp.bfloat16), w2_ref[...].astype(jnp.bfloat16),
                    preferred_element_type=jnp.float32)
            + b2_ref[...]
        )

    dinv = _dinv_of(deg_ref)
    h = jnp.dot(z_scr[...], wc_ref[...], preferred_element_type=jnp.float32)
    o_ref[0] = h * dinv[:, None]


def _encg(x, w1, b1, w2, b2, wc, deg2):
    blk = 1000
    return pl.pallas_call(
        _encg_body,
        grid=(N_NODES // blk, NC),
        in_specs=[
            pl.BlockSpec((blk, IN_DIM), lambda i, j: (i, 0)),
            pl.BlockSpec((IN_DIM, IN_DIM), lambda i, j: (0, 0)),
            pl.BlockSpec((1, IN_DIM), lambda i, j: (0, 0)),
            pl.BlockSpec((IN_DIM, EMBED_DIM), lambda i, j: (0, 0)),
            pl.BlockSpec((1, EMBED_DIM), lambda i, j: (0, 0)),
            pl.BlockSpec((EMBED_DIM, HALF), lambda i, j: (0, j)),
            pl.BlockSpec((blk, NC), lambda i, j: (i, 0)),
        ],
        out_specs=pl.BlockSpec((1, blk, HALF), lambda i, j: (j, i, 0)),
        out_shape=jax.ShapeDtypeStruct((NC, N_NODES, HALF), jnp.float32),
        scratch_shapes=[pltpu.VMEM((blk, EMBED_DIM), jnp.float32)],
    )(x, w1, b1, w2, b2, wc, deg2)


def _mid_body(t0_ref, t1_ref, deg_ref, w_ref, b_ref, o_ref):
    dinv = _dinv_of(deg_ref)
    t = jnp.concatenate([t0_ref[0], t1_ref[0]], axis=-1)
    a = t * dinv[:, None] + b_ref[...]
    r = jnp.maximum(a, 0.0)
    h = jnp.dot(r, w_ref[...], preferred_element_type=jnp.float32)
    o_ref[0] = h * dinv[:, None]


def _mid(t, deg2, w, b):
    blk = 1000
    return pl.pallas_call(
        _mid_body,
        grid=(N_NODES // blk, NC),
        in_specs=[
            pl.BlockSpec((1, blk, HALF), lambda i, j: (0, i, 0)),
            pl.BlockSpec((1, blk, HALF), lambda i, j: (1, i, 0)),
            pl.BlockSpec((blk, NC), lambda i, j: (i, 0)),
            pl.BlockSpec((EMBED_DIM, HALF), lambda i, j: (0, j)),
            pl.BlockSpec((1, EMBED_DIM), lambda i, j: (0, 0)),
        ],
        out_specs=pl.BlockSpec((1, blk, HALF), lambda i, j: (j, i, 0)),
        out_shape=jax.ShapeDtypeStruct((NC, N_NODES, HALF), jnp.float32),
    )(t, t, deg2, w, b)


def _final_body(t0_ref, t1_ref, deg_ref, b_ref, o_ref):
    dinv = _dinv_of(deg_ref)
    t = jnp.concatenate([t0_ref[0], t1_ref[0]], axis=-1)
    o_ref[...] = t * dinv[:, None] + b_ref[...]


def _final(t, deg2, b):
    blk = 1000
    return pl.pallas_call(
        _final_body,
        grid=(N_NODES // blk,),
        in_specs=[
            pl.BlockSpec((1, blk, HALF), lambda i: (0, i, 0)),
            pl.BlockSpec((1, blk, HALF), lambda i: (1, i, 0)),
            pl.BlockSpec((blk, NC), lambda i: (i, 0)),
            pl.BlockSpec((1, EMBED_DIM), lambda i: (0, 0)),
        ],
        out_specs=pl.BlockSpec((blk, EMBED_DIM), lambda i: (i, 0)),
        out_shape=jax.ShapeDtypeStruct((N_NODES, EMBED_DIM), jnp.float32),
    )(t, t, deg2, b)


def _loss_body(x_ref, y_ref, o_ref):
    i = pl.program_id(0)
    x = x_ref[...]
    y = y_ref[...]
    sx = jnp.sum(x * x, axis=1)
    sy = jnp.sum(y * y, axis=1)
    d = jnp.sum(x * y, axis=1)
    ix = 1.0 / jnp.maximum(jnp.sqrt(sx), 1e-12)
    iy = 1.0 / jnp.maximum(jnp.sqrt(sy), 1e-12)
    term = sx * ix * ix + sy * iy * iy - 2.0 * d * ix * iy
    p = (jnp.sum(term) * (1.0 / PAIRS)).reshape(1, 1)

    @pl.when(i == 0)
    def _():
        o_ref[...] = p

    @pl.when(i > 0)
    def _():
        o_ref[...] += p


def _loss(xy):
    blk = 512
    nblk = PAIRS // blk
    return pl.pallas_call(
        _loss_body,
        grid=(nblk,),
        in_specs=[
            pl.BlockSpec((blk, EMBED_DIM), lambda i: (i, 0)),
            pl.BlockSpec((blk, EMBED_DIM), lambda i: (i + nblk, 0)),
        ],
        out_specs=pl.BlockSpec((1, 1), lambda i: (0, 0)),
        out_shape=jax.ShapeDtypeStruct((1, 1), jnp.float32),
    )(xy, xy)


@functools.partial(
    pl.kernel,
    out_type=jax.ShapeDtypeStruct((NC * N_NODES,), jnp.float32),
    mesh=_sc_mesh,
    scratch_types=[
        pltpu.VMEM((CHUNK,), jnp.int32),
        pltpu.VMEM((CHUNK,), jnp.int32),
        pltpu.VMEM((CHUNK,), jnp.int32),
        pltpu.VMEM((CHUNK,), jnp.int32),
        pltpu.VMEM((CHUNK,), jnp.float32),
        pltpu.VMEM((DEG_ACC,), jnp.float32),
        pltpu.VMEM_SHARED((DEG_ACC,), jnp.float32),
        pltpu.SemaphoreType.DMA,
        pltpu.SemaphoreType.DMA,
        pltpu.SemaphoreType.DMA,
        pltpu.SemaphoreType.DMA,
        pltpu.SemaphoreType.DMA,
        pltpu.SemaphoreType.DMA,
    ],
)
def _deg_kernel(dst_hbm, zeros_hbm, ones_hbm, deg_out,
                i0, i1, i2, i3, ones_v, stage_v, acc_sh,
                s0, s1, s2, s3, c0, c1):
    c = lax.axis_index("c")
    s = lax.axis_index("s")
    base = (c * NS + s) * EPT_DEG
    ibufs = (i0, i1, i2, i3)
    isems = (s0, s1, s2, s3)
    csems = (c0, c1)

    @pl.when(s == 0)
    def _():
        pltpu.sync_copy(zeros_hbm, stage_v)
        pltpu.sync_copy(stage_v, acc_sh)

    pltpu.sync_copy(ones_hbm, ones_v)
    plsc.subcore_barrier()

    def _fire_idx(j, b):
        pltpu.async_copy(dst_hbm.at[pl.ds(base + j * CHUNK, CHUNK)], ibufs[b], isems[b])

    def _wait_idx(j, b):
        pltpu.make_async_copy(dst_hbm.at[pl.ds(base + j * CHUNK, CHUNK)], ibufs[b], isems[b]).wait()

    def _fire_scat(b):
        pltpu.async_copy(ones_v, acc_sh.at[ibufs[b]], csems[b % 2], add=True)

    def _wait_scat(b):
        pltpu.make_async_copy(ones_v, acc_sh.at[ibufs[b]], csems[b % 2]).wait()

    _fire_idx(0, 0)
    _fire_idx(1, 1)

    def body(m, carry):
        for b in range(4):
            j = 4 * m + b
            if b == 0:
                @pl.when(j > 0)
                def _():
                    _wait_scat(3)
            else:
                _wait_scat(b - 1)
            _wait_idx(j, b)
            _fire_scat(b)
            if b < 2:
                _fire_idx(j + 2, (b + 2) % 4)
            else:
                @pl.when(j + 2 < DEG_N)
                def _():
                    _fire_idx(j + 2, (b + 2) % 4)
        return carry

    lax.fori_loop(0, DEG_N // 4, body, 0)
    _wait_scat(3)
    plsc.subcore_barrier()

    @pl.when(s == 0)
    def _():
        pltpu.sync_copy(acc_sh, stage_v)
        pltpu.sync_copy(stage_v.at[pl.ds(0, N_NODES)],
                        deg_out.at[pl.ds(c * N_NODES, N_NODES)])


@functools.partial(
    pl.kernel,
    out_type=jax.ShapeDtypeStruct((NC, N_NODES, HALF), jnp.float32),
    mesh=_sc_mesh,
    scratch_types=[
        pltpu.VMEM((CHUNK,), jnp.int32),
        pltpu.VMEM((CHUNK,), jnp.int32),
        pltpu.VMEM((CHUNK,), jnp.int32),
        pltpu.VMEM((CHUNK,), jnp.int32),
        pltpu.VMEM((CHUNK,), jnp.int32),
        pltpu.VMEM((CHUNK,), jnp.int32),
        pltpu.VMEM((2, CHUNK, HALF), jnp.float32),
        pltpu.VMEM_SHARED((ACC_ROWS, HALF), jnp.float32),
        pltpu.SemaphoreType.DMA,
        pltpu.SemaphoreType.DMA,
        pltpu.SemaphoreType.DMA,
        pltpu.SemaphoreType.DMA,
        pltpu.SemaphoreType.DMA,
        pltpu.SemaphoreType.DMA,
        pltpu.SemaphoreType.DMA,
        pltpu.SemaphoreType.DMA,
        pltpu.SemaphoreType.DMA,
        pltpu.SemaphoreType.DMA,
    ],
)
def _agg_kernel(gflat_hbm, srcf_hbm, dst_hbm, t_out,
                sidx0, sidx1, didx0, didx1, didx2, didx3, rows_v, acc_sh,
                ss0, ss1, ds0, ds1, ds2, ds3, rs0, rs1, cs0, cs1):
    c = lax.axis_index("c")
    s = lax.axis_index("s")
    rbase = s * ROWS_PER_TILE
    pltpu.sync_copy(
        gflat_hbm.at[pl.ds(c * GROWS + rbase, ROWS_PER_TILE)],
        acc_sh.at[pl.ds(rbase, ROWS_PER_TILE)],
    )

    @pl.when(s == 0)
    def _():
        pltpu.sync_copy(
            gflat_hbm.at[pl.ds(c * GROWS + NS * ROWS_PER_TILE, ROWS_REM)],
            acc_sh.at[pl.ds(NS * ROWS_PER_TILE, ROWS_REM)],
        )

    plsc.subcore_barrier()

    sbufs = (sidx0, sidx1)
    ssems = (ss0, ss1)
    dbufs = (didx0, didx1, didx2, didx3)
    dsems = (ds0, ds1, ds2, ds3)
    rsems = (rs0, rs1)
    csems = (cs0, cs1)
    sbase = c * EROWS * CHUNK + s * NCHUNK * CHUNK
    dbase = s * NCHUNK * CHUNK

    def _fire_is(j, b):
        pltpu.async_copy(srcf_hbm.at[pl.ds(sbase + j * CHUNK, CHUNK)], sbufs[b], ssems[b])

    def _wait_is(j, b):
        pltpu.make_async_copy(srcf_hbm.at[pl.ds(sbase + j * CHUNK, CHUNK)], sbufs[b], ssems[b]).wait()

    def _fire_id(j, b):
        pltpu.async_copy(dst_hbm.at[pl.ds(dbase + j * CHUNK, CHUNK)], dbufs[b], dsems[b])

    def _wait_id(j, b):
        pltpu.make_async_copy(dst_hbm.at[pl.ds(dbase + j * CHUNK, CHUNK)], dbufs[b], dsems[b]).wait()

    def _fire_row(rb):
        pltpu.async_copy(gflat_hbm.at[sbufs[rb]], rows_v.at[rb], rsems[rb])

    def _wait_row(rb):
        pltpu.make_async_copy(gflat_hbm.at[sbufs[rb]], rows_v.at[rb], rsems[rb]).wait()

    def _fire_scat(rb, db):
        pltpu.async_copy(rows_v.at[rb], acc_sh.at[dbufs[db]], csems[rb], add=True)

    def _wait_scat(rb, db):
        pltpu.make_async_copy(rows_v.at[rb], acc_sh.at[dbufs[db]], csems[rb]).wait()

    _fire_is(0, 0)
    _fire_id(0, 0)
    _fire_is(1, 1)
    _fire_id(1, 1)
    _wait_is(0, 0)
    _fire_row(0)

    def body(m, carry):
        for b in range(4):
            j = 4 * m + b
            rb = b % 2
            _wait_row(rb)
            if b == 0:
                @pl.when(j > 0)
                def _():
                    _wait_scat(1 - rb, 3)
            else:
                _wait_scat(1 - rb, b - 1)
            if b == 3:
                @pl.when(j + 1 < NCHUNK)
                def _():
                    _wait_is(j + 1, 1 - rb)
                    _fire_row(1 - rb)
            else:
                _wait_is(j + 1, 1 - rb)
                _fire_row(1 - rb)
            _wait_id(j, b)
            _fire_scat(rb, b)
            if b < 2:
                _fire_is(j + 2, rb)
                _fire_id(j + 2, (b + 2) % 4)
            else:
                @pl.when(j + 2 < NCHUNK)
                def _():
                    _fire_is(j + 2, rb)
                    _fire_id(j + 2, (b + 2) % 4)
        return carry

    lax.fori_loop(0, NCHUNK // 4, body, 0)
    _wait_scat(1, 3)
    plsc.subcore_barrier()
    pltpu.sync_copy(
        acc_sh.at[pl.ds(rbase, ROWS_PER_TILE)],
        t_out.at[c, pl.ds(rbase, ROWS_PER_TILE)],
    )

    @pl.when(s == 0)
    def _():
        pltpu.sync_copy(
            acc_sh.at[pl.ds(NS * ROWS_PER_TILE, ROWS_REM)],
            t_out.at[c, pl.ds(NS * ROWS_PER_TILE, ROWS_REM)],
        )


@functools.partial(
    pl.kernel,
    out_type=jax.ShapeDtypeStruct((2 * PAIRS, EMBED_DIM), jnp.float32),
    mesh=_sc_mesh,
    scratch_types=[
        pltpu.VMEM((CHUNK,), jnp.int32),
        pltpu.VMEM((CHUNK, EMBED_DIM), jnp.float32),
    ],
)
def _pair_gather(emb_hbm, idx_hbm, out_hbm, idx_v, rows_v):
    c = lax.axis_index("c")
    s = lax.axis_index("s")
    w = c * NS + s

    def body(k, carry):
        b = w * GPT + k * CHUNK
        pltpu.sync_copy(idx_hbm.at[pl.ds(b, CHUNK)], idx_v)
        pltpu.sync_copy(emb_hbm.at[idx_v], rows_v)
        pltpu.sync_copy(rows_v, out_hbm.at[pl.ds(b, CHUNK)])
        return carry

    lax.fori_loop(0, GPT // CHUNK, body, 0)


def kernel(inputs, edge_index, item_emb, attr_emb,
           enc_W1, enc_b1, enc_W2, enc_b2,
           conv1_W, conv1_b, conv2_W, conv2_b):
    x = jnp.concatenate([item_emb, attr_emb], axis=0)
    pad_ids = jnp.arange(E_PAD, dtype=jnp.int32)
    src = jnp.concatenate(
        [edge_index[0].astype(jnp.int32), N_NODES + pad_ids % ZROWS])
    dst = jnp.concatenate(
        [edge_index[1].astype(jnp.int32), (pad_ids * 13) % N_NODES])
    dstdeg = jnp.concatenate(
        [edge_index[1].astype(jnp.int32), N_NODES + pad_ids % 8])
    src2 = jnp.concatenate([src, src + GROWS])
    idx_pairs = jnp.transpose(inputs).reshape(-1).astype(jnp.int32)
    zeros = jnp.zeros((DEG_ACC,), jnp.float32)
    ones = jnp.ones((CHUNK,), jnp.float32)

    deg2 = _deg_kernel(dstdeg, zeros, ones).reshape(NC, N_NODES).T
    g1 = _encg(x, enc_W1, enc_b1.reshape(1, -1), enc_W2, enc_b2.reshape(1, -1),
               conv1_W, deg2)
    zpad = jnp.zeros((NC, ZROWS, HALF), jnp.float32)

    def _gpad(g):
        return jnp.concatenate([g, zpad], axis=1).reshape(NC * GROWS, HALF)

    t1 = _agg_kernel(_gpad(g1), src2, dst)
    g2 = _mid(t1, deg2, conv2_W, conv1_b.reshape(1, -1))
    t2 = _agg_kernel(_gpad(g2), src2, dst)
    emb = _final(t2, deg2, conv2_b.reshape(1, -1))
    xy = _pair_gather(emb, idx_pairs)
    loss = _loss(xy)[0, 0]
    return (loss, emb)


# speedup vs baseline: 1.0242x; 1.0242x over previous
import functools

import jax
import jax.numpy as jnp
from jax import lax
from jax.experimental import pallas as pl
from jax.experimental.pallas import tpu as pltpu
from jax.experimental.pallas import tpu_sc as plsc

N_ITEMS = 8000
N_ATTRS = 2000
N_NODES = N_ITEMS + N_ATTRS
N_EDGES = 160000
IN_DIM = 512
EMBED_DIM = 256
HALF = EMBED_DIM // 2

NC = 2
NS = 16
CHUNK = 128

EROWS = 1280
E_PAD = EROWS * CHUNK - N_EDGES
TROWS = EROWS // NS
NCHUNK = TROWS
ACC_ROWS = N_NODES
ZROWS = 1000
GROWS = N_NODES + ZROWS

DEG_EDGES = EROWS * CHUNK
EPT_DEG = DEG_EDGES // (NC * NS)
DEG_N = EPT_DEG // CHUNK
DEG_ACC = N_NODES + 8

ROWS_PER_TILE = 624
ROWS_REM = N_NODES - NS * ROWS_PER_TILE
PAIRS = 4096
GPT = 2 * PAIRS // (NC * NS)

_sc_mesh = plsc.VectorSubcoreMesh(core_axis_name="c", subcore_axis_name="s")


def _dinv_of(deg_ref):
    return lax.rsqrt(deg_ref[:, 0] + deg_ref[:, 1] + 1.0)


def _encg_body(x_ref, w1_ref, b1_ref, w2_ref, b2_ref, wc_ref, deg_ref,
               o_ref, z_scr):
    i = pl.program_id(0)
    j = pl.program_id(1)

    @pl.when(j == 0)
    def _():
        h = jnp.dot(x_ref[...].astype(jnp.bfloat16),
                    w1_ref[...].astype(jnp.bfloat16),
                    preferred_element_type=jnp.float32)
        h = h + b1_ref[...]
        h = jnp.where(h > 0, h, jnp.exp(h) - 1.0)
        z_scr[...] = (
            jnp.dot(h.astype(jnp.bfloat16), w2_ref[...].astype(jnp.bfloat16),
                    preferred_element_type=jnp.float32)
            + b2_ref[...]
        )

    dinv = _dinv_of(deg_ref)
    h = jnp.dot(z_scr[...], wc_ref[...], preferred_element_type=jnp.float32)
    g = h * dinv[:, None]
    o_ref[0] = jnp.where(i < N_NODES // 1000, g, 0.0)


def _encg(x, w1, b1, w2, b2, wc, deg2):
    blk = 1000
    nreal = N_NODES // blk
    return pl.pallas_call(
        _encg_body,
        grid=(GROWS // blk, NC),
        in_specs=[
            pl.BlockSpec((blk, IN_DIM), lambda i, j: (jnp.minimum(i, nreal - 1), 0)),
            pl.BlockSpec((IN_DIM, IN_DIM), lambda i, j: (0, 0)),
            pl.BlockSpec((1, IN_DIM), lambda i, j: (0, 0)),
            pl.BlockSpec((IN_DIM, EMBED_DIM), lambda i, j: (0, 0)),
            pl.BlockSpec((1, EMBED_DIM), lambda i, j: (0, 0)),
            pl.BlockSpec((EMBED_DIM, HALF), lambda i, j: (0, j)),
            pl.BlockSpec((blk, NC), lambda i, j: (jnp.minimum(i, nreal - 1), 0)),
        ],
        out_specs=pl.BlockSpec((1, blk, HALF), lambda i, j: (j, i, 0)),
        out_shape=jax.ShapeDtypeStruct((NC, GROWS, HALF), jnp.float32),
        scratch_shapes=[pltpu.VMEM((blk, EMBED_DIM), jnp.float32)],
    )(x, w1, b1, w2, b2, wc, deg2)


def _mid_body(t0_ref, t1_ref, deg_ref, w_ref, b_ref, o_ref):
    i = pl.program_id(0)
    dinv = _dinv_of(deg_ref)
    t = jnp.concatenate([t0_ref[0], t1_ref[0]], axis=-1)
    a = t * dinv[:, None] + b_ref[...]
    r = jnp.maximum(a, 0.0)
    h = jnp.dot(r, w_ref[...], preferred_element_type=jnp.float32)
    g = h * dinv[:, None]
    o_ref[0] = jnp.where(i < N_NODES // 1000, g, 0.0)


def _mid(t, deg2, w, b):
    blk = 1000
    nreal = N_NODES // blk
    return pl.pallas_call(
        _mid_body,
        grid=(GROWS // blk, NC),
        in_specs=[
            pl.BlockSpec((1, blk, HALF), lambda i, j: (0, jnp.minimum(i, nreal - 1), 0)),
            pl.BlockSpec((1, blk, HALF), lambda i, j: (1, jnp.minimum(i, nreal - 1), 0)),
            pl.BlockSpec((blk, NC), lambda i, j: (jnp.minimum(i, nreal - 1), 0)),
            pl.BlockSpec((EMBED_DIM, HALF), lambda i, j: (0, j)),
            pl.BlockSpec((1, EMBED_DIM), lambda i, j: (0, 0)),
        ],
        out_specs=pl.BlockSpec((1, blk, HALF), lambda i, j: (j, i, 0)),
        out_shape=jax.ShapeDtypeStruct((NC, GROWS, HALF), jnp.float32),
    )(t, t, deg2, w, b)


def _final_body(t0_ref, t1_ref, deg_ref, b_ref, o_ref):
    dinv = _dinv_of(deg_ref)
    t = jnp.concatenate([t0_ref[0], t1_ref[0]], axis=-1)
    o_ref[...] = t * dinv[:, None] + b_ref[...]


def _final(t, deg2, b):
    blk = 1000
    return pl.pallas_call(
        _final_body,
        grid=(N_NODES // blk,),
        in_specs=[
            pl.BlockSpec((1, blk, HALF), lambda i: (0, i, 0)),
            pl.BlockSpec((1, blk, HALF), lambda i: (1, i, 0)),
            pl.BlockSpec((blk, NC), lambda i: (i, 0)),
            pl.BlockSpec((1, EMBED_DIM), lambda i: (0, 0)),
        ],
        out_specs=pl.BlockSpec((blk, EMBED_DIM), lambda i: (i, 0)),
        out_shape=jax.ShapeDtypeStruct((N_NODES, EMBED_DIM), jnp.float32),
    )(t, t, deg2, b)


def _loss_body(x_ref, y_ref, o_ref):
    i = pl.program_id(0)
    x = x_ref[...]
    y = y_ref[...]
    sx = jnp.sum(x * x, axis=1)
    sy = jnp.sum(y * y, axis=1)
    d = jnp.sum(x * y, axis=1)
    ix = 1.0 / jnp.maximum(jnp.sqrt(sx), 1e-12)
    iy = 1.0 / jnp.maximum(jnp.sqrt(sy), 1e-12)
    term = sx * ix * ix + sy * iy * iy - 2.0 * d * ix * iy
    p = (jnp.sum(term) * (1.0 / PAIRS)).reshape(1, 1)

    @pl.when(i == 0)
    def _():
        o_ref[...] = p

    @pl.when(i > 0)
    def _():
        o_ref[...] += p


def _loss(xy):
    blk = 512
    nblk = PAIRS // blk
    return pl.pallas_call(
        _loss_body,
        grid=(nblk,),
        in_specs=[
            pl.BlockSpec((blk, EMBED_DIM), lambda i: (i, 0)),
            pl.BlockSpec((blk, EMBED_DIM), lambda i: (i + nblk, 0)),
        ],
        out_specs=pl.BlockSpec((1, 1), lambda i: (0, 0)),
        out_shape=jax.ShapeDtypeStruct((1, 1), jnp.float32),
    )(xy, xy)


@functools.partial(
    pl.kernel,
    out_type=jax.ShapeDtypeStruct((NC * N_NODES,), jnp.float32),
    mesh=_sc_mesh,
    scratch_types=[
        pltpu.VMEM((CHUNK,), jnp.int32),
        pltpu.VMEM((CHUNK,), jnp.int32),
        pltpu.VMEM((CHUNK,), jnp.int32),
        pltpu.VMEM((CHUNK,), jnp.int32),
        pltpu.VMEM((CHUNK,), jnp.float32),
        pltpu.VMEM((DEG_ACC,), jnp.float32),
        pltpu.VMEM_SHARED((DEG_ACC,), jnp.float32),
        pltpu.SemaphoreType.DMA,
        pltpu.SemaphoreType.DMA,
        pltpu.SemaphoreType.DMA,
        pltpu.SemaphoreType.DMA,
        pltpu.SemaphoreType.DMA,
        pltpu.SemaphoreType.DMA,
    ],
)
def _deg_kernel(dst_hbm, zeros_hbm, ones_hbm, deg_out,
                i0, i1, i2, i3, ones_v, stage_v, acc_sh,
                s0, s1, s2, s3, c0, c1):
    c = lax.axis_index("c")
    s = lax.axis_index("s")
    base = (c * NS + s) * EPT_DEG
    ibufs = (i0, i1, i2, i3)
    isems = (s0, s1, s2, s3)
    csems = (c0, c1)

    @pl.when(s == 0)
    def _():
        pltpu.sync_copy(zeros_hbm, stage_v)
        pltpu.sync_copy(stage_v, acc_sh)

    pltpu.sync_copy(ones_hbm, ones_v)
    plsc.subcore_barrier()

    def _fire_idx(j, b):
        pltpu.async_copy(dst_hbm.at[pl.ds(base + j * CHUNK, CHUNK)], ibufs[b], isems[b])

    def _wait_idx(j, b):
        pltpu.make_async_copy(dst_hbm.at[pl.ds(base + j * CHUNK, CHUNK)], ibufs[b], isems[b]).wait()

    def _fire_scat(b):
        pltpu.async_copy(ones_v, acc_sh.at[ibufs[b]], csems[b % 2], add=True)

    def _wait_scat(b):
        pltpu.make_async_copy(ones_v, acc_sh.at[ibufs[b]], csems[b % 2]).wait()

    _fire_idx(0, 0)
    _fire_idx(1, 1)

    def body(m, carry):
        for b in range(4):
            j = 4 * m + b
            if b == 0:
                @pl.when(j > 0)
                def _():
                    _wait_scat(3)
            else:
                _wait_scat(b - 1)
            _wait_idx(j, b)
            _fire_scat(b)
            if b < 2:
                _fire_idx(j + 2, (b + 2) % 4)
            else:
                @pl.when(j + 2 < DEG_N)
                def _():
                    _fire_idx(j + 2, (b + 2) % 4)
        return carry

    lax.fori_loop(0, DEG_N // 4, body, 0)
    _wait_scat(3)
    plsc.subcore_barrier()

    @pl.when(s == 0)
    def _():
        pltpu.sync_copy(acc_sh, stage_v)
        pltpu.sync_copy(stage_v.at[pl.ds(0, N_NODES)],
                        deg_out.at[pl.ds(c * N_NODES, N_NODES)])


@functools.partial(
    pl.kernel,
    out_type=jax.ShapeDtypeStruct((NC, N_NODES, HALF), jnp.float32),
    mesh=_sc_mesh,
    scratch_types=[
        pltpu.VMEM((CHUNK,), jnp.int32),
        pltpu.VMEM((CHUNK,), jnp.int32),
        pltpu.VMEM((CHUNK,), jnp.int32),
        pltpu.VMEM((CHUNK,), jnp.int32),
        pltpu.VMEM((CHUNK,), jnp.int32),
        pltpu.VMEM((CHUNK,), jnp.int32),
        pltpu.VMEM((2, CHUNK, HALF), jnp.float32),
        pltpu.VMEM_SHARED((ACC_ROWS, HALF), jnp.float32),
        pltpu.SemaphoreType.DMA,
        pltpu.SemaphoreType.DMA,
        pltpu.SemaphoreType.DMA,
        pltpu.SemaphoreType.DMA,
        pltpu.SemaphoreType.DMA,
        pltpu.SemaphoreType.DMA,
        pltpu.SemaphoreType.DMA,
        pltpu.SemaphoreType.DMA,
        pltpu.SemaphoreType.DMA,
        pltpu.SemaphoreType.DMA,
    ],
)
def _agg_kernel(gflat_hbm, srcf_hbm, dst_hbm, t_out,
                sidx0, sidx1, didx0, didx1, didx2, didx3, rows_v, acc_sh,
                ss0, ss1, ds0, ds1, ds2, ds3, rs0, rs1, cs0, cs1):
    c = lax.axis_index("c")
    s = lax.axis_index("s")
    rbase = s * ROWS_PER_TILE
    pltpu.sync_copy(
        gflat_hbm.at[pl.ds(c * GROWS + rbase, ROWS_PER_TILE)],
        acc_sh.at[pl.ds(rbase, ROWS_PER_TILE)],
    )

    @pl.when(s == 0)
    def _():
        pltpu.sync_copy(
            gflat_hbm.at[pl.ds(c * GROWS + NS * ROWS_PER_TILE, ROWS_REM)],
            acc_sh.at[pl.ds(NS * ROWS_PER_TILE, ROWS_REM)],
        )

    plsc.subcore_barrier()

    sbufs = (sidx0, sidx1)
    ssems = (ss0, ss1)
    dbufs = (didx0, didx1, didx2, didx3)
    dsems = (ds0, ds1, ds2, ds3)
    rsems = (rs0, rs1)
    csems = (cs0, cs1)
    sbase = c * EROWS * CHUNK + s * NCHUNK * CHUNK
    dbase = s * NCHUNK * CHUNK

    def _fire_is(j, b):
        pltpu.async_copy(srcf_hbm.at[pl.ds(sbase + j * CHUNK, CHUNK)], sbufs[b], ssems[b])

    def _wait_is(j, b):
        pltpu.make_async_copy(srcf_hbm.at[pl.ds(sbase + j * CHUNK, CHUNK)], sbufs[b], ssems[b]).wait()

    def _fire_id(j, b):
        pltpu.async_copy(dst_hbm.at[pl.ds(dbase + j * CHUNK, CHUNK)], dbufs[b], dsems[b])

    def _wait_id(j, b):
        pltpu.make_async_copy(dst_hbm.at[pl.ds(dbase + j * CHUNK, CHUNK)], dbufs[b], dsems[b]).wait()

    def _fire_row(rb):
        pltpu.async_copy(gflat_hbm.at[sbufs[rb]], rows_v.at[rb], rsems[rb])

    def _wait_row(rb):
        pltpu.make_async_copy(gflat_hbm.at[sbufs[rb]], rows_v.at[rb], rsems[rb]).wait()

    def _fire_scat(rb, db):
        pltpu.async_copy(rows_v.at[rb], acc_sh.at[dbufs[db]], csems[rb], add=True)

    def _wait_scat(rb, db):
        pltpu.make_async_copy(rows_v.at[rb], acc_sh.at[dbufs[db]], csems[rb]).wait()

    _fire_is(0, 0)
    _fire_id(0, 0)
    _fire_is(1, 1)
    _fire_id(1, 1)
    _wait_is(0, 0)
    _fire_row(0)

    def body(m, carry):
        for b in range(4):
            j = 4 * m + b
            rb = b % 2
            _wait_row(rb)
            if b == 0:
                @pl.when(j > 0)
                def _():
                    _wait_scat(1 - rb, 3)
            else:
                _wait_scat(1 - rb, b - 1)
            if b == 3:
                @pl.when(j + 1 < NCHUNK)
                def _():
                    _wait_is(j + 1, 1 - rb)
                    _fire_row(1 - rb)
            else:
                _wait_is(j + 1, 1 - rb)
                _fire_row(1 - rb)
            _wait_id(j, b)
            _fire_scat(rb, b)
            if b < 2:
                _fire_is(j + 2, rb)
                _fire_id(j + 2, (b + 2) % 4)
            else:
                @pl.when(j + 2 < NCHUNK)
                def _():
                    _fire_is(j + 2, rb)
                    _fire_id(j + 2, (b + 2) % 4)
        return carry

    lax.fori_loop(0, NCHUNK // 4, body, 0)
    _wait_scat(1, 3)
    plsc.subcore_barrier()
    pltpu.sync_copy(
        acc_sh.at[pl.ds(rbase, ROWS_PER_TILE)],
        t_out.at[c, pl.ds(rbase, ROWS_PER_TILE)],
    )

    @pl.when(s == 0)
    def _():
        pltpu.sync_copy(
            acc_sh.at[pl.ds(NS * ROWS_PER_TILE, ROWS_REM)],
            t_out.at[c, pl.ds(NS * ROWS_PER_TILE, ROWS_REM)],
        )


@functools.partial(
    pl.kernel,
    out_type=jax.ShapeDtypeStruct((2 * PAIRS, EMBED_DIM), jnp.float32),
    mesh=_sc_mesh,
    scratch_types=[
        pltpu.VMEM((CHUNK,), jnp.int32),
        pltpu.VMEM((CHUNK, EMBED_DIM), jnp.float32),
    ],
)
def _pair_gather(emb_hbm, idx_hbm, out_hbm, idx_v, rows_v):
    c = lax.axis_index("c")
    s = lax.axis_index("s")
    w = c * NS + s

    def body(k, carry):
        b = w * GPT + k * CHUNK
        pltpu.sync_copy(idx_hbm.at[pl.ds(b, CHUNK)], idx_v)
        pltpu.sync_copy(emb_hbm.at[idx_v], rows_v)
        pltpu.sync_copy(rows_v, out_hbm.at[pl.ds(b, CHUNK)])
        return carry

    lax.fori_loop(0, GPT // CHUNK, body, 0)


def kernel(inputs, edge_index, item_emb, attr_emb,
           enc_W1, enc_b1, enc_W2, enc_b2,
           conv1_W, conv1_b, conv2_W, conv2_b):
    x = jnp.concatenate([item_emb, attr_emb], axis=0)
    pad_ids = jnp.arange(E_PAD, dtype=jnp.int32)
    src = jnp.concatenate(
        [edge_index[0].astype(jnp.int32), N_NODES + pad_ids % ZROWS])
    dst = jnp.concatenate(
        [edge_index[1].astype(jnp.int32), (pad_ids * 13) % N_NODES])
    dstdeg = jnp.concatenate(
        [edge_index[1].astype(jnp.int32), N_NODES + pad_ids % 8])
    src2 = jnp.concatenate([src, src + GROWS])
    idx_pairs = jnp.transpose(inputs).reshape(-1).astype(jnp.int32)
    zeros = jnp.zeros((DEG_ACC,), jnp.float32)
    ones = jnp.ones((CHUNK,), jnp.float32)

    deg2 = _deg_kernel(dstdeg, zeros, ones).reshape(NC, N_NODES).T
    g1 = _encg(x, enc_W1, enc_b1.reshape(1, -1), enc_W2, enc_b2.reshape(1, -1),
               conv1_W, deg2)
    t1 = _agg_kernel(g1.reshape(NC * GROWS, HALF), src2, dst)
    g2 = _mid(t1, deg2, conv2_W, conv1_b.reshape(1, -1))
    t2 = _agg_kernel(g2.reshape(NC * GROWS, HALF), src2, dst)
    emb = _final(t2, deg2, conv2_b.reshape(1, -1))
    xy = _pair_gather(emb, idx_pairs)
    loss = _loss(xy)[0, 0]
    return (loss, emb)
